# baseline (speedup 1.0000x reference)
import jax
import jax.numpy as jnp
from jax.experimental import pallas as pl
from jax.experimental.pallas import tpu as pltpu

_BATCH = 262144
_CLASS_NUM = 40
_N = _BATCH * _CLASS_NUM
_BLOCK_B = 4096
_HALF_BLOCKS = (_BATCH // 2) // _BLOCK_B
_CHUNK = 128


def _bce_term_sum(p_ref, t_ref, acc):
    def body(j, acc):
        p = p_ref[pl.ds(j * _CHUNK, _CHUNK), :]
        t = t_ref[pl.ds(j * _CHUNK, _CHUNK), :]
        a = jnp.abs(p)
        u = jnp.exp2(-1.4426950408889634 * a)
        term = 0.5 * (p + a) - p * t + 0.6931471805599453 * jnp.log2(1.0 + u)
        return acc + jnp.sum(term.reshape(_CHUNK // 8, 8, _CLASS_NUM), axis=0)

    return jax.lax.fori_loop(0, _BLOCK_B // _CHUNK, body, acc)


def _bce_sum_kernel(p0_ref, t0_ref, p1_ref, t1_ref, out_ref, acc_ref):
    i = pl.program_id(0)

    @pl.when(i == 0)
    def _init():
        acc_ref[...] = jnp.zeros_like(acc_ref)

    acc = jnp.zeros((8, _CLASS_NUM), jnp.float32)
    acc = _bce_term_sum(p0_ref, t0_ref, acc)
    acc = _bce_term_sum(p1_ref, t1_ref, acc)
    acc_ref[...] += acc

    @pl.when(i == pl.num_programs(0) - 1)
    def _finalize():
        out_ref[0] = jnp.sum(acc_ref[...]) / _N


def kernel(pred, target):
    lo = pl.BlockSpec((_BLOCK_B, _CLASS_NUM), lambda i: (i, 0))
    hi = pl.BlockSpec((_BLOCK_B, _CLASS_NUM), lambda i: (i + _HALF_BLOCKS, 0))
    out = pl.pallas_call(
        _bce_sum_kernel,
        grid=(_HALF_BLOCKS,),
        in_specs=[lo, lo, hi, hi],
        out_specs=pl.BlockSpec(
            (1,), lambda i: (0,), memory_space=pltpu.SMEM
        ),
        out_shape=jax.ShapeDtypeStruct((1,), jnp.float32),
        scratch_shapes=[pltpu.VMEM((8, _CLASS_NUM), jnp.float32)],
    )(pred, target, pred, target)
    return out[0]
